# baseline (speedup 1.0000x reference)
import jax
import jax.numpy as jnp
from jax.experimental import pallas as pl

N = 10000
NUM_GRAPHS = 16


def _gine(x, ea, src, dst, g, We, be, Wg, bg, Wa, ba, Wb, bb):
    m = jax.nn.relu(x[src] + ea @ We + be)
    aggr = jax.ops.segment_sum(m, dst, num_segments=x.shape[0])
    gt = g @ Wg + bg
    out = aggr + x + gt[None, :]
    h = jax.nn.relu(out @ Wa + ba)
    return h @ Wb + bb


def _head_kernel(pooled_ref, w_ref, b_ref, o_ref):
    o_ref[...] = jax.nn.sigmoid(pooled_ref[...] @ w_ref[...] + b_ref[...]) * 0.5


def kernel(x, edge_index, edge_attr, xA, dosd_distances, batch, params):
    p = params
    src, dst = edge_index[0], edge_index[1]
    dosd_vals = dosd_distances[src, dst][:, None]
    ea = jnp.concatenate([edge_attr, dosd_vals], axis=1)
    h = x
    for l in (1, 2, 3):
        h = jax.nn.relu(_gine(h, ea, src, dst, xA,
                              p[f"We{l}"], p[f"be{l}"],
                              p[f"Wg{l}"], p[f"bg{l}"],
                              p[f"W{l}a"], p[f"b{l}a"],
                              p[f"W{l}b"], p[f"b{l}b"]))
        h = h @ p[f"Wf{l}"] + p[f"bf{l}"]
        if l < 3:
            h = jax.nn.relu(h)
    sums = jax.ops.segment_sum(h, batch, num_segments=NUM_GRAPHS)
    cnt = jax.ops.segment_sum(jnp.ones((h.shape[0],), jnp.float32), batch,
                              num_segments=NUM_GRAPHS)
    pooled = sums / jnp.clip(cnt, 1.0)[:, None]
    out = pl.pallas_call(
        _head_kernel,
        out_shape=jax.ShapeDtypeStruct((NUM_GRAPHS, 1), jnp.float32),
    )(pooled, p["Wfc"], p["bfc"])
    return out


# speedup vs baseline: 2.8835x; 2.8835x over previous
import functools

import jax
import jax.numpy as jnp
from jax import lax
from jax.experimental import pallas as pl
from jax.experimental.pallas import tpu as pltpu
from jax.experimental.pallas import tpu_sc as plsc

N = 10000
E = 320000
NNFEAT = 128
H = 256
OUT = 1024
NUM_GRAPHS = 16

NC = 2
NS = 16
NW = NC * NS
L = 16

K = 80
EPW = E // NW
EPS = E // NS
NPS = N // NS
ZROWS = 125
DL = 128

_MESH = plsc.VectorSubcoreMesh(core_axis_name="c", subcore_axis_name="s")


def _dosd_body(rows_hbm, ridx_hbm, cidx_hbm, out_hbm, ridx, cidx, rowbuf,
               outbuf, sem):
    c = lax.axis_index("c")
    s = lax.axis_index("s")
    wid = s * NC + c

    def chunk(j, _):
        base = wid * EPW + j * K
        pltpu.sync_copy(ridx_hbm.at[pl.ds(base, K)], ridx)
        pltpu.sync_copy(cidx_hbm.at[pl.ds(base, K)], cidx)
        pltpu.async_copy(rows_hbm.at[ridx], rowbuf, sem).wait()
        for i in range(K // L):
            eidx = jnp.arange(L, dtype=jnp.int32) + i * L
            cols = cidx[pl.ds(i * L, L)]
            outbuf[pl.ds(i * L, L)] = plsc.load_gather(rowbuf, [eidx, cols])
        pltpu.sync_copy(outbuf, out_hbm.at[pl.ds(base, K)])
        return 0

    lax.fori_loop(0, EPW // K, chunk, 0)


_dosd_gather = functools.partial(
    pl.kernel,
    out_type=jax.ShapeDtypeStruct((E,), jnp.float32),
    mesh=_MESH,
    compiler_params=pltpu.CompilerParams(needs_layout_passes=False),
    scratch_types=[
        pltpu.VMEM((K,), jnp.int32),
        pltpu.VMEM((K,), jnp.int32),
        pltpu.VMEM((K, DL), jnp.float32),
        pltpu.VMEM((K,), jnp.float32),
        pltpu.SemaphoreType.DMA,
    ],
)(_dosd_body)


HC = 128
WR = 624


def _edge_body(esplit, h_hbm, eaw_hbm, src_hbm, dst_hbm, out_hbm, sidx, didx,
               gidx, hbuf, ebuf, zbuf, aggr_sh, sem):
    c = lax.axis_index("c")
    s = lax.axis_index("s")

    def zrow(r, _):
        for cc in range(HC // L):
            zbuf[r, pl.ds(cc * L, L)] = jnp.zeros((L,), jnp.float32)
        return 0

    lax.fori_loop(0, ZROWS, zrow, 0)

    def zcopy(j, _):
        pltpu.sync_copy(zbuf, aggr_sh.at[pl.ds(s * NPS + j * ZROWS, ZROWS)])
        return 0

    lax.fori_loop(0, NPS // ZROWS, zcopy, 0)
    plsc.subcore_barrier()

    if esplit:
        base0 = (c * NS + s) * EPW
        goff = 0
        eoff = 0
        nchunks = EPW // K
    else:
        base0 = s * EPS
        goff = c * N
        eoff = c * E
        nchunks = EPS // K

    def chunk(j, _):
        base = base0 + j * K
        pltpu.sync_copy(src_hbm.at[pl.ds(base, K)], sidx)
        pltpu.sync_copy(dst_hbm.at[pl.ds(base, K)], didx)
        for i in range(K // L):
            gidx[pl.ds(i * L, L)] = sidx[pl.ds(i * L, L)] + goff
        pltpu.async_copy(h_hbm.at[gidx], hbuf, sem).wait()
        pltpu.sync_copy(eaw_hbm.at[pl.ds(eoff + base, K)], ebuf)

        def row(r, _):
            for cc in range(HC // L):
                sl = pl.ds(cc * L, L)
                ebuf[r, sl] = jnp.maximum(hbuf[r, sl] + ebuf[r, sl], 0.0)
            return 0

        lax.fori_loop(0, K, row, 0)
        pltpu.sync_copy(ebuf, aggr_sh.at[didx], add=True)
        return 0

    lax.fori_loop(0, nchunks, chunk, 0)
    plsc.subcore_barrier()
    pltpu.sync_copy(aggr_sh.at[pl.ds(s * WR, WR)],
                    out_hbm.at[pl.ds(c * N + s * WR, WR)])

    @pl.when(s == 0)
    def _():
        pltpu.sync_copy(aggr_sh.at[pl.ds(NS * WR, N - NS * WR)],
                        out_hbm.at[pl.ds(c * N + NS * WR, N - NS * WR)])


@functools.cache
def _edge_stage(esplit):
    return functools.partial(
        pl.kernel,
        out_type=jax.ShapeDtypeStruct((2 * N, HC), jnp.float32),
        mesh=_MESH,
        scratch_types=[
            pltpu.VMEM((K,), jnp.int32),
            pltpu.VMEM((K,), jnp.int32),
            pltpu.VMEM((K,), jnp.int32),
            pltpu.VMEM((K, HC), jnp.float32),
            pltpu.VMEM((K, HC), jnp.float32),
            pltpu.VMEM((ZROWS, HC), jnp.float32),
            pltpu.VMEM_SHARED((N, HC), jnp.float32),
            pltpu.SemaphoreType.DMA,
        ],
    )(functools.partial(_edge_body, esplit))


EB = 2000


def _eaw_body(ea_ref, dv_ref, w1_ref, l1_ref, b1_ref, w2_ref, l2_ref, b2_ref,
              w3_ref, l3_ref, b3_ref, o1_ref, o2_ref, o3_ref):
    ea = ea_ref[...]
    dv = dv_ref[...]
    for w_ref, lw_ref, b_ref, o_ref, split in (
            (w1_ref, l1_ref, b1_ref, o1_ref, False),
            (w2_ref, l2_ref, b2_ref, o2_ref, True),
            (w3_ref, l3_ref, b3_ref, o3_ref, True)):
        res = (jnp.dot(ea, w_ref[...], preferred_element_type=jnp.float32)
               + dv * lw_ref[...] + b_ref[...])
        if split:
            o_ref[0] = res[:, :HC]
            o_ref[1] = res[:, HC:]
        else:
            o_ref[...] = res


def _eaw_all(edge_attr, dosd_vals, params):
    p = params
    wspecs = []
    wargs = []
    for l in (1, 2, 3):
        we = p[f"We{l}"]
        cin = we.shape[1]
        wargs += [we[:17], we[17].reshape(1, cin), p[f"be{l}"].reshape(1, cin)]
        wspecs += [pl.BlockSpec((17, cin), lambda i: (0, 0)),
                   pl.BlockSpec((1, cin), lambda i: (0, 0)),
                   pl.BlockSpec((1, cin), lambda i: (0, 0))]
    return pl.pallas_call(
        _eaw_body,
        grid=(E // EB,),
        in_specs=[pl.BlockSpec((EB, 17), lambda i: (i, 0)),
                  pl.BlockSpec((EB, 1), lambda i: (i, 0))] + wspecs,
        out_specs=[pl.BlockSpec((EB, 128), lambda i: (i, 0)),
                   pl.BlockSpec((2, EB, 128), lambda i: (0, i, 0)),
                   pl.BlockSpec((2, EB, 128), lambda i: (0, i, 0))],
        out_shape=[jax.ShapeDtypeStruct((E, 128), jnp.float32),
                   jax.ShapeDtypeStruct((2, E, 128), jnp.float32),
                   jax.ShapeDtypeStruct((2, E, 128), jnp.float32)],
    )(edge_attr, dosd_vals, *wargs)


NB = 1000


def _node_body(first, last, alo_ref, ahi_ref, hlo_ref, hhi_ref, xa_ref,
               wg_ref, bg_ref, wa_ref, ba_ref, wb_ref, bb_ref, wf_ref,
               bf_ref, o_ref):
    gt = jnp.dot(xa_ref[...], wg_ref[...],
                 preferred_element_type=jnp.float32) + bg_ref[...]
    if first:
        inp = alo_ref[...] + ahi_ref[...] + hlo_ref[...] + gt
    else:
        inp = jnp.concatenate(
            [alo_ref[...] + hlo_ref[...], ahi_ref[...] + hhi_ref[...]],
            axis=1) + gt
    t1 = jax.nn.relu(jnp.dot(inp, wa_ref[...],
                             preferred_element_type=jnp.float32) + ba_ref[...])
    t2 = jax.nn.relu(jnp.dot(t1, wb_ref[...],
                             preferred_element_type=jnp.float32) + bb_ref[...])
    h3 = jnp.dot(t2, wf_ref[...],
                 preferred_element_type=jnp.float32) + bf_ref[...]
    if not last:
        h3 = jax.nn.relu(h3)
        o_ref[0] = h3[:, :128]
        o_ref[1] = h3[:, 128:]
    else:
        o_ref[...] = h3


def _node_mlp(l, aggr_flat, h_flat, xA, params):
    p = params
    cin = NNFEAT if l == 1 else H
    first = l == 1
    hcin = cin if first else cin // 2
    last = l == 3
    nblk = N // NB
    args = [aggr_flat, aggr_flat, h_flat, h_flat,
            xA.reshape(1, 21), p[f"Wg{l}"], p[f"bg{l}"].reshape(1, cin),
            p[f"W{l}a"], p[f"b{l}a"].reshape(1, H),
            p[f"W{l}b"], p[f"b{l}b"].reshape(1, OUT),
            p[f"Wf{l}"], p[f"bf{l}"].reshape(1, H)]
    in_specs = [
        pl.BlockSpec((NB, HC), lambda i: (i, 0)),
        pl.BlockSpec((NB, HC), lambda i: (i + nblk, 0)),
        pl.BlockSpec((NB, hcin), lambda i: (i, 0)),
        pl.BlockSpec((NB, hcin), lambda i: (i, 0) if first
                     else (i + nblk, 0)),
        pl.BlockSpec((1, 21), lambda i: (0, 0)),
        pl.BlockSpec((21, cin), lambda i: (0, 0)),
        pl.BlockSpec((1, cin), lambda i: (0, 0)),
        pl.BlockSpec((cin, H), lambda i: (0, 0)),
        pl.BlockSpec((1, H), lambda i: (0, 0)),
        pl.BlockSpec((H, OUT), lambda i: (0, 0)),
        pl.BlockSpec((1, OUT), lambda i: (0, 0)),
        pl.BlockSpec((OUT, H), lambda i: (0, 0)),
        pl.BlockSpec((1, H), lambda i: (0, 0)),
    ]
    if last:
        out_spec = pl.BlockSpec((NB, H), lambda i: (i, 0))
        out_shape = jax.ShapeDtypeStruct((N, H), jnp.float32)
    else:
        out_spec = pl.BlockSpec((2, NB, 128), lambda i: (0, i, 0))
        out_shape = jax.ShapeDtypeStruct((2, N, 128), jnp.float32)
    return pl.pallas_call(
        functools.partial(_node_body, first, last),
        grid=(nblk,),
        in_specs=in_specs,
        out_specs=out_spec,
        out_shape=out_shape,
    )(*args)


def _pool_body(h_ref, b_ref, ps_ref, cnt_ref):
    i = pl.program_id(0)
    mask = (lax.broadcasted_iota(jnp.int32, (NUM_GRAPHS, NB), 0)
            == b_ref[0]).astype(jnp.float32)
    ps = jnp.dot(mask, h_ref[...], preferred_element_type=jnp.float32)
    cnt = jnp.sum(mask, axis=1, keepdims=True)

    @pl.when(i == 0)
    def _():
        ps_ref[...] = jnp.zeros_like(ps_ref)
        cnt_ref[...] = jnp.zeros_like(cnt_ref)

    ps_ref[...] += ps
    cnt_ref[...] += cnt


def _pool(h, batch_row):
    return pl.pallas_call(
        _pool_body,
        grid=(N // NB,),
        in_specs=[pl.BlockSpec((NB, H), lambda i: (i, 0)),
                  pl.BlockSpec((1, 1, NB), lambda i: (i, 0, 0))],
        out_specs=[pl.BlockSpec((NUM_GRAPHS, H), lambda i: (0, 0)),
                   pl.BlockSpec((NUM_GRAPHS, 1), lambda i: (0, 0))],
        out_shape=[jax.ShapeDtypeStruct((NUM_GRAPHS, H), jnp.float32),
                   jax.ShapeDtypeStruct((NUM_GRAPHS, 1), jnp.float32)],
    )(h, batch_row)


def _head_body(ps_ref, cnt_ref, w_ref, b_ref, o_ref):
    pooled = ps_ref[...] / jnp.maximum(cnt_ref[...], 1.0)
    o_ref[...] = jax.nn.sigmoid(
        jnp.dot(pooled, w_ref[...], preferred_element_type=jnp.float32)
        + b_ref[...]) * 0.5


def _head(ps, cnt, wfc, bfc):
    return pl.pallas_call(
        _head_body,
        out_shape=jax.ShapeDtypeStruct((NUM_GRAPHS, 1), jnp.float32),
    )(ps, cnt, wfc, bfc.reshape(1, 1))


def kernel(x, edge_index, edge_attr, xA, dosd_distances, batch, params):
    p = params
    src = edge_index[0]
    dst = edge_index[1]

    flat = src * N + dst
    row_idx = flat // DL
    col_idx = flat % DL
    dosd_rows = dosd_distances.reshape(N * N // DL, DL)
    dosd_vals = _dosd_gather(dosd_rows, row_idx, col_idx)

    eaw1, eaw2, eaw3 = _eaw_all(edge_attr, dosd_vals.reshape(E, 1), p)

    batch_row = batch.reshape(N // NB, 1, NB)

    h_flat = x
    for l, eaw in ((1, eaw1), (2, eaw2), (3, eaw3)):
        esplit = l == 1
        aggr_flat = _edge_stage(esplit)(h_flat, eaw.reshape(-1, HC), src, dst)
        if l < 3:
            h_flat = _node_mlp(l, aggr_flat, h_flat, xA, p).reshape(2 * N, HC)
        else:
            h_final = _node_mlp(l, aggr_flat, h_flat, xA, p)
    ps, cnt = _pool(h_final, batch_row)
    return _head(ps, cnt, p["Wfc"], p["bfc"])


# speedup vs baseline: 5.1496x; 1.7858x over previous
import functools

import jax
import jax.numpy as jnp
from jax import lax
from jax.experimental import pallas as pl
from jax.experimental.pallas import tpu as pltpu
from jax.experimental.pallas import tpu_sc as plsc

N = 10000
E = 320000
NNFEAT = 128
H = 256
OUT = 1024
NUM_GRAPHS = 16

NC = 2
NS = 16
NW = NC * NS
L = 16

K = 80
EPW = E // NW
EPS = E // NS
NPS = N // NS
ZROWS = 125
DL = 128

_MESH = plsc.VectorSubcoreMesh(core_axis_name="c", subcore_axis_name="s")


def _dosd_body(rows_hbm, ridx_hbm, cidx_hbm, out_hbm, ridx, cidx, rowbuf,
               outbuf, sem):
    c = lax.axis_index("c")
    s = lax.axis_index("s")
    wid = s * NC + c

    def chunk(j, _):
        base = wid * EPW + j * K
        pltpu.sync_copy(ridx_hbm.at[pl.ds(base, K)], ridx)
        pltpu.sync_copy(cidx_hbm.at[pl.ds(base, K)], cidx)
        pltpu.async_copy(rows_hbm.at[ridx], rowbuf, sem).wait()
        for i in range(K // L):
            eidx = jnp.arange(L, dtype=jnp.int32) + i * L
            cols = cidx[pl.ds(i * L, L)]
            outbuf[pl.ds(i * L, L)] = plsc.load_gather(rowbuf, [eidx, cols])
        pltpu.sync_copy(outbuf, out_hbm.at[pl.ds(base, K)])
        return 0

    lax.fori_loop(0, EPW // K, chunk, 0)


_dosd_gather = functools.partial(
    pl.kernel,
    out_type=jax.ShapeDtypeStruct((E,), jnp.float32),
    mesh=_MESH,
    compiler_params=pltpu.CompilerParams(needs_layout_passes=False),
    scratch_types=[
        pltpu.VMEM((K,), jnp.int32),
        pltpu.VMEM((K,), jnp.int32),
        pltpu.VMEM((K, DL), jnp.float32),
        pltpu.VMEM((K,), jnp.float32),
        pltpu.SemaphoreType.DMA,
    ],
)(_dosd_body)


HC = 128
WR = 624


def _edge_body(esplit, h_hbm, eaw_hbm, src_hbm, dst_hbm, out_hbm,
               sidx0, didx0, gidx0, dscat0, hbuf0, ebuf0,
               sidx1, didx1, gidx1, dscat1, hbuf1, ebuf1,
               aggr_sh,
               isem0, gsem0, esem0, ssem0, isem1, gsem1, esem1, ssem1):
    c = lax.axis_index("c")
    s = lax.axis_index("s")

    def zrow(r, _):
        for cc in range(HC // L):
            ebuf0[r, pl.ds(cc * L, L)] = jnp.zeros((L,), jnp.float32)
        return 0

    lax.fori_loop(0, K, zrow, 0)

    def zcopy(j, _):
        pltpu.sync_copy(ebuf0, aggr_sh.at[pl.ds(s * NPS + j * K, K)])
        return 0

    lax.fori_loop(0, NPS // K, zcopy, 0)
    pltpu.sync_copy(ebuf0.at[pl.ds(0, NPS - (NPS // K) * K)],
                    aggr_sh.at[pl.ds(s * NPS + (NPS // K) * K,
                                     NPS - (NPS // K) * K)])
    plsc.subcore_barrier()

    if esplit:
        base0 = (c * NS + s) * EPW
        goff = 0
        eoff = 0
        nchunks = EPW // K
    else:
        base0 = s * EPS
        goff = c * N
        eoff = c * E
        nchunks = EPS // K

    sets = ((sidx0, didx0, gidx0, dscat0, hbuf0, ebuf0, isem0, gsem0, esem0,
             ssem0),
            (sidx1, didx1, gidx1, dscat1, hbuf1, ebuf1, isem1, gsem1, esem1,
             ssem1))

    def issue_idx(j, p):
        b = base0 + j * K
        si, di, _, _, _, _, ise, _, _, _ = sets[p]
        pltpu.async_copy(src_hbm.at[pl.ds(b, K)], si, ise)
        pltpu.async_copy(dst_hbm.at[pl.ds(b, K)], di, ise)

    def wait_idx(p):
        si, di, _, _, _, _, ise, _, _, _ = sets[p]
        pltpu.make_async_copy(src_hbm.at[pl.ds(0, K)], si, ise).wait()
        pltpu.make_async_copy(dst_hbm.at[pl.ds(0, K)], di, ise).wait()

    def issue_fetch(j, p):
        b = base0 + j * K
        si, di, gi, dsc, hb, eb, _, gse, ese, _ = sets[p]
        for i in range(K // L):
            sl = pl.ds(i * L, L)
            gi[sl] = si[sl] + goff
            dsc[sl] = di[sl]
        pltpu.async_copy(h_hbm.at[gi], hb, gse)
        pltpu.async_copy(eaw_hbm.at[pl.ds(eoff + b, K)], eb, ese)

    def wait_fetch(p):
        _, _, gi, _, hb, eb, _, gse, ese, _ = sets[p]
        pltpu.make_async_copy(h_hbm.at[gi], hb, gse).wait()
        pltpu.make_async_copy(eaw_hbm.at[pl.ds(0, K)], eb, ese).wait()

    def compute(p):
        hb, eb = sets[p][4], sets[p][5]

        def row(r, _):
            for cc in range(HC // L):
                sl = pl.ds(cc * L, L)
                eb[r, sl] = jnp.maximum(hb[r, sl] + eb[r, sl], 0.0)
            return 0

        lax.fori_loop(0, K, row, 0)

    def issue_scatter(p):
        _, _, _, dsc, _, eb, _, _, _, sse = sets[p]
        pltpu.async_copy(eb, aggr_sh.at[dsc], sse, add=True)

    def wait_scatter(p):
        _, _, _, dsc, _, eb, _, _, _, sse = sets[p]
        pltpu.make_async_copy(eb, aggr_sh.at[dsc], sse).wait()

    def body(j, p, static_last=False):
        pbar = 1 - p

        if not static_last:
            @pl.when(j + 1 < nchunks)
            def _():
                pl.when(j >= 1)(lambda: wait_scatter(pbar))
                wait_idx(pbar)
                issue_fetch(j + 1, pbar)
                pl.when(j + 2 < nchunks)(lambda: issue_idx(j + 2, p))

        wait_fetch(p)
        compute(p)
        issue_scatter(p)

    issue_idx(0, 0)
    issue_idx(1, 1)
    wait_idx(0)
    issue_fetch(0, 0)

    def pair(t, _):
        body(t * 2, 0)
        body(t * 2 + 1, 1)
        return 0

    lax.fori_loop(0, nchunks // 2, pair, 0)
    if nchunks % 2:
        body(nchunks - 1, 0, static_last=True)
    wait_scatter(nchunks % 2)
    wait_scatter((nchunks + 1) % 2)
    plsc.subcore_barrier()
    pltpu.sync_copy(aggr_sh.at[pl.ds(s * WR, WR)],
                    out_hbm.at[pl.ds(c * N + s * WR, WR)])

    @pl.when(s == 0)
    def _():
        pltpu.sync_copy(aggr_sh.at[pl.ds(NS * WR, N - NS * WR)],
                        out_hbm.at[pl.ds(c * N + NS * WR, N - NS * WR)])


@functools.cache
def _edge_stage(esplit):
    return functools.partial(
        pl.kernel,
        out_type=jax.ShapeDtypeStruct((2 * N, HC), jnp.float32),
        mesh=_MESH,
        scratch_types=(
            [pltpu.VMEM((K,), jnp.int32)] * 4
            + [pltpu.VMEM((K, HC), jnp.float32)] * 2
            + [pltpu.VMEM((K,), jnp.int32)] * 4
            + [pltpu.VMEM((K, HC), jnp.float32)] * 2
            + [pltpu.VMEM_SHARED((N, HC), jnp.float32)]
            + [pltpu.SemaphoreType.DMA] * 8
        ),
    )(functools.partial(_edge_body, esplit))


EB = 2000


def _eaw_body(ea_ref, dv_ref, w1_ref, l1_ref, b1_ref, w2_ref, l2_ref, b2_ref,
              w3_ref, l3_ref, b3_ref, o1_ref, o2_ref, o3_ref):
    ea = ea_ref[...]
    dv = dv_ref[...]
    for w_ref, lw_ref, b_ref, o_ref, split in (
            (w1_ref, l1_ref, b1_ref, o1_ref, False),
            (w2_ref, l2_ref, b2_ref, o2_ref, True),
            (w3_ref, l3_ref, b3_ref, o3_ref, True)):
        res = (jnp.dot(ea, w_ref[...], preferred_element_type=jnp.float32)
               + dv * lw_ref[...] + b_ref[...])
        if split:
            o_ref[0] = res[:, :HC]
            o_ref[1] = res[:, HC:]
        else:
            o_ref[...] = res


def _eaw_all(edge_attr, dosd_vals, params):
    p = params
    wspecs = []
    wargs = []
    for l in (1, 2, 3):
        we = p[f"We{l}"]
        cin = we.shape[1]
        wargs += [we[:17], we[17].reshape(1, cin), p[f"be{l}"].reshape(1, cin)]
        wspecs += [pl.BlockSpec((17, cin), lambda i: (0, 0)),
                   pl.BlockSpec((1, cin), lambda i: (0, 0)),
                   pl.BlockSpec((1, cin), lambda i: (0, 0))]
    return pl.pallas_call(
        _eaw_body,
        grid=(E // EB,),
        in_specs=[pl.BlockSpec((EB, 17), lambda i: (i, 0)),
                  pl.BlockSpec((EB, 1), lambda i: (i, 0))] + wspecs,
        out_specs=[pl.BlockSpec((EB, 128), lambda i: (i, 0)),
                   pl.BlockSpec((2, EB, 128), lambda i: (0, i, 0)),
                   pl.BlockSpec((2, EB, 128), lambda i: (0, i, 0))],
        out_shape=[jax.ShapeDtypeStruct((E, 128), jnp.float32),
                   jax.ShapeDtypeStruct((2, E, 128), jnp.float32),
                   jax.ShapeDtypeStruct((2, E, 128), jnp.float32)],
    )(edge_attr, dosd_vals, *wargs)


NB = 1000


def _node_body(first, last, alo_ref, ahi_ref, hlo_ref, hhi_ref, xa_ref,
               wg_ref, bg_ref, wa_ref, ba_ref, wb_ref, bb_ref, wf_ref,
               bf_ref, o_ref):
    gt = jnp.dot(xa_ref[...], wg_ref[...],
                 preferred_element_type=jnp.float32) + bg_ref[...]
    if first:
        inp = alo_ref[...] + ahi_ref[...] + hlo_ref[...] + gt
    else:
        inp = jnp.concatenate(
            [alo_ref[...] + hlo_ref[...], ahi_ref[...] + hhi_ref[...]],
            axis=1) + gt
    t1 = jax.nn.relu(jnp.dot(inp, wa_ref[...],
                             preferred_element_type=jnp.float32) + ba_ref[...])
    t2 = jax.nn.relu(jnp.dot(t1, wb_ref[...],
                             preferred_element_type=jnp.float32) + bb_ref[...])
    h3 = jnp.dot(t2, wf_ref[...],
                 preferred_element_type=jnp.float32) + bf_ref[...]
    if not last:
        h3 = jax.nn.relu(h3)
        o_ref[0] = h3[:, :128]
        o_ref[1] = h3[:, 128:]
    else:
        o_ref[...] = h3


def _node_mlp(l, aggr_flat, h_flat, xA, params):
    p = params
    cin = NNFEAT if l == 1 else H
    first = l == 1
    hcin = cin if first else cin // 2
    last = l == 3
    nblk = N // NB
    args = [aggr_flat, aggr_flat, h_flat, h_flat,
            xA.reshape(1, 21), p[f"Wg{l}"], p[f"bg{l}"].reshape(1, cin),
            p[f"W{l}a"], p[f"b{l}a"].reshape(1, H),
            p[f"W{l}b"], p[f"b{l}b"].reshape(1, OUT),
            p[f"Wf{l}"], p[f"bf{l}"].reshape(1, H)]
    in_specs = [
        pl.BlockSpec((NB, HC), lambda i: (i, 0)),
        pl.BlockSpec((NB, HC), lambda i: (i + nblk, 0)),
        pl.BlockSpec((NB, hcin), lambda i: (i, 0)),
        pl.BlockSpec((NB, hcin), lambda i: (i, 0) if first
                     else (i + nblk, 0)),
        pl.BlockSpec((1, 21), lambda i: (0, 0)),
        pl.BlockSpec((21, cin), lambda i: (0, 0)),
        pl.BlockSpec((1, cin), lambda i: (0, 0)),
        pl.BlockSpec((cin, H), lambda i: (0, 0)),
        pl.BlockSpec((1, H), lambda i: (0, 0)),
        pl.BlockSpec((H, OUT), lambda i: (0, 0)),
        pl.BlockSpec((1, OUT), lambda i: (0, 0)),
        pl.BlockSpec((OUT, H), lambda i: (0, 0)),
        pl.BlockSpec((1, H), lambda i: (0, 0)),
    ]
    if last:
        out_spec = pl.BlockSpec((NB, H), lambda i: (i, 0))
        out_shape = jax.ShapeDtypeStruct((N, H), jnp.float32)
    else:
        out_spec = pl.BlockSpec((2, NB, 128), lambda i: (0, i, 0))
        out_shape = jax.ShapeDtypeStruct((2, N, 128), jnp.float32)
    return pl.pallas_call(
        functools.partial(_node_body, first, last),
        grid=(nblk,),
        in_specs=in_specs,
        out_specs=out_spec,
        out_shape=out_shape,
    )(*args)


def _pool_body(h_ref, b_ref, ps_ref, cnt_ref):
    i = pl.program_id(0)
    mask = (lax.broadcasted_iota(jnp.int32, (NUM_GRAPHS, NB), 0)
            == b_ref[0]).astype(jnp.float32)
    ps = jnp.dot(mask, h_ref[...], preferred_element_type=jnp.float32)
    cnt = jnp.sum(mask, axis=1, keepdims=True)

    @pl.when(i == 0)
    def _():
        ps_ref[...] = jnp.zeros_like(ps_ref)
        cnt_ref[...] = jnp.zeros_like(cnt_ref)

    ps_ref[...] += ps
    cnt_ref[...] += cnt


def _pool(h, batch_row):
    return pl.pallas_call(
        _pool_body,
        grid=(N // NB,),
        in_specs=[pl.BlockSpec((NB, H), lambda i: (i, 0)),
                  pl.BlockSpec((1, 1, NB), lambda i: (i, 0, 0))],
        out_specs=[pl.BlockSpec((NUM_GRAPHS, H), lambda i: (0, 0)),
                   pl.BlockSpec((NUM_GRAPHS, 1), lambda i: (0, 0))],
        out_shape=[jax.ShapeDtypeStruct((NUM_GRAPHS, H), jnp.float32),
                   jax.ShapeDtypeStruct((NUM_GRAPHS, 1), jnp.float32)],
    )(h, batch_row)


def _head_body(ps_ref, cnt_ref, w_ref, b_ref, o_ref):
    pooled = ps_ref[...] / jnp.maximum(cnt_ref[...], 1.0)
    o_ref[...] = jax.nn.sigmoid(
        jnp.dot(pooled, w_ref[...], preferred_element_type=jnp.float32)
        + b_ref[...]) * 0.5


def _head(ps, cnt, wfc, bfc):
    return pl.pallas_call(
        _head_body,
        out_shape=jax.ShapeDtypeStruct((NUM_GRAPHS, 1), jnp.float32),
    )(ps, cnt, wfc, bfc.reshape(1, 1))


def kernel(x, edge_index, edge_attr, xA, dosd_distances, batch, params):
    p = params
    src = edge_index[0]
    dst = edge_index[1]

    flat = src * N + dst
    row_idx = flat // DL
    col_idx = flat % DL
    dosd_rows = dosd_distances.reshape(N * N // DL, DL)
    dosd_vals = _dosd_gather(dosd_rows, row_idx, col_idx)

    eaw1, eaw2, eaw3 = _eaw_all(edge_attr, dosd_vals.reshape(E, 1), p)

    batch_row = batch.reshape(N // NB, 1, NB)

    h_flat = x
    for l, eaw in ((1, eaw1), (2, eaw2), (3, eaw3)):
        esplit = l == 1
        aggr_flat = _edge_stage(esplit)(h_flat, eaw.reshape(-1, HC), src, dst)
        if l < 3:
            h_flat = _node_mlp(l, aggr_flat, h_flat, xA, p).reshape(2 * N, HC)
        else:
            h_final = _node_mlp(l, aggr_flat, h_flat, xA, p)
    ps, cnt = _pool(h_final, batch_row)
    return _head(ps, cnt, p["Wfc"], p["bfc"])


# speedup vs baseline: 5.6043x; 1.0883x over previous
import functools

import jax
import jax.numpy as jnp
from jax import lax
from jax.experimental import pallas as pl
from jax.experimental.pallas import tpu as pltpu
from jax.experimental.pallas import tpu_sc as plsc

N = 10000
E = 320000
NNFEAT = 128
H = 256
OUT = 1024
NUM_GRAPHS = 16

NC = 2
NS = 16
NW = NC * NS
L = 16

K = 80
EPW = E // NW
EPS = E // NS
NPS = N // NS

_MESH = plsc.VectorSubcoreMesh(core_axis_name="c", subcore_axis_name="s")


def _dosd_body(tab_hbm, fidx_hbm, out_hbm, fidx0, fidx1, vbuf0, vbuf1,
               isem0, isem1, gsem0, gsem1):
    c = lax.axis_index("c")
    s = lax.axis_index("s")
    wid = s * NC + c
    base0 = wid * EPW
    sets = ((fidx0, vbuf0, isem0, gsem0), (fidx1, vbuf1, isem1, gsem1))

    def issue_idx(j, p):
        fi, _, ise, _ = sets[p]
        pltpu.async_copy(fidx_hbm.at[pl.ds(base0 + j * K, K)], fi, ise)

    def issue_gather(p):
        fi, vb, ise, gse = sets[p]
        pltpu.make_async_copy(fidx_hbm.at[pl.ds(0, K)], fi, ise).wait()
        pltpu.async_copy(tab_hbm.at[fi], vb, gse)

    def drain(j, p):
        fi, vb, _, gse = sets[p]
        pltpu.make_async_copy(tab_hbm.at[fi], vb, gse).wait()
        pltpu.sync_copy(vb, out_hbm.at[pl.ds(base0 + j * K, K)])

    issue_idx(0, 0)
    issue_idx(1, 1)
    issue_gather(0)

    def pair(t, _):
        j = t * 2

        def half(j, p):
            pl.when(j + 1 < EPW // K)(lambda: issue_gather(1 - p))
            drain(j, p)
            pl.when(j + 2 < EPW // K)(lambda: issue_idx(j + 2, p))

        half(j, 0)
        half(j + 1, 1)
        return 0

    lax.fori_loop(0, EPW // (2 * K), pair, 0)
    if (EPW // K) % 2:
        drain(EPW // K - 1, 0)


_dosd_gather = functools.partial(
    pl.kernel,
    out_type=jax.ShapeDtypeStruct((E,), jnp.float32),
    mesh=_MESH,
    scratch_types=[
        pltpu.VMEM((K,), jnp.int32),
        pltpu.VMEM((K,), jnp.int32),
        pltpu.VMEM((K,), jnp.float32),
        pltpu.VMEM((K,), jnp.float32),
        pltpu.SemaphoreType.DMA,
        pltpu.SemaphoreType.DMA,
        pltpu.SemaphoreType.DMA,
        pltpu.SemaphoreType.DMA,
    ],
)(_dosd_body)


HC = 128
WR = 624


def _edge_body(esplit, h_hbm, eaw_hbm, src_hbm, dst_hbm, out_hbm,
               sidx0, didx0, gidx0, dscat0, hbuf0, ebuf0,
               sidx1, didx1, gidx1, dscat1, hbuf1, ebuf1,
               aggr_sh,
               isem0, gsem0, esem0, ssem0, isem1, gsem1, esem1, ssem1):
    c = lax.axis_index("c")
    s = lax.axis_index("s")

    def zrow(r, _):
        for cc in range(HC // L):
            ebuf0[r, pl.ds(cc * L, L)] = jnp.zeros((L,), jnp.float32)
        return 0

    lax.fori_loop(0, K, zrow, 0)

    def zcopy(j, _):
        pltpu.sync_copy(ebuf0, aggr_sh.at[pl.ds(s * NPS + j * K, K)])
        return 0

    lax.fori_loop(0, NPS // K, zcopy, 0)
    pltpu.sync_copy(ebuf0.at[pl.ds(0, NPS - (NPS // K) * K)],
                    aggr_sh.at[pl.ds(s * NPS + (NPS // K) * K,
                                     NPS - (NPS // K) * K)])
    plsc.subcore_barrier()

    if esplit:
        base0 = (c * NS + s) * EPW
        goff = 0
        eoff = 0
        nchunks = EPW // K
    else:
        base0 = s * EPS
        goff = c * N
        eoff = c * E
        nchunks = EPS // K

    sets = ((sidx0, didx0, gidx0, dscat0, hbuf0, ebuf0, isem0, gsem0, esem0,
             ssem0),
            (sidx1, didx1, gidx1, dscat1, hbuf1, ebuf1, isem1, gsem1, esem1,
             ssem1))

    def issue_idx(j, p):
        b = base0 + j * K
        si, di, _, _, _, _, ise, _, _, _ = sets[p]
        pltpu.async_copy(src_hbm.at[pl.ds(b, K)], si, ise)
        pltpu.async_copy(dst_hbm.at[pl.ds(b, K)], di, ise)

    def wait_idx(p):
        si, di, _, _, _, _, ise, _, _, _ = sets[p]
        pltpu.make_async_copy(src_hbm.at[pl.ds(0, K)], si, ise).wait()
        pltpu.make_async_copy(dst_hbm.at[pl.ds(0, K)], di, ise).wait()

    def issue_fetch(j, p):
        b = base0 + j * K
        si, di, gi, dsc, hb, eb, _, gse, ese, _ = sets[p]
        for i in range(K // L):
            sl = pl.ds(i * L, L)
            gi[sl] = si[sl] + goff
            dsc[sl] = di[sl]
        pltpu.async_copy(h_hbm.at[gi], hb, gse)
        pltpu.async_copy(eaw_hbm.at[pl.ds(eoff + b, K)], eb, ese)

    def wait_fetch(p):
        _, _, gi, _, hb, eb, _, gse, ese, _ = sets[p]
        pltpu.make_async_copy(h_hbm.at[gi], hb, gse).wait()
        pltpu.make_async_copy(eaw_hbm.at[pl.ds(0, K)], eb, ese).wait()

    def compute(p):
        hb, eb = sets[p][4], sets[p][5]

        def row(r, _):
            for cc in range(HC // L):
                sl = pl.ds(cc * L, L)
                eb[r, sl] = jnp.maximum(hb[r, sl] + eb[r, sl], 0.0)
            return 0

        lax.fori_loop(0, K, row, 0)

    def issue_scatter(p):
        _, _, _, dsc, _, eb, _, _, _, sse = sets[p]
        pltpu.async_copy(eb, aggr_sh.at[dsc], sse, add=True)

    def wait_scatter(p):
        _, _, _, dsc, _, eb, _, _, _, sse = sets[p]
        pltpu.make_async_copy(eb, aggr_sh.at[dsc], sse).wait()

    def body(j, p, static_last=False):
        pbar = 1 - p

        if not static_last:
            @pl.when(j + 1 < nchunks)
            def _():
                pl.when(j >= 1)(lambda: wait_scatter(pbar))
                wait_idx(pbar)
                issue_fetch(j + 1, pbar)
                pl.when(j + 2 < nchunks)(lambda: issue_idx(j + 2, p))

        wait_fetch(p)
        compute(p)
        issue_scatter(p)

    issue_idx(0, 0)
    issue_idx(1, 1)
    wait_idx(0)
    issue_fetch(0, 0)

    def pair(t, _):
        body(t * 2, 0)
        body(t * 2 + 1, 1)
        return 0

    lax.fori_loop(0, nchunks // 2, pair, 0)
    if nchunks % 2:
        body(nchunks - 1, 0, static_last=True)
    wait_scatter(nchunks % 2)
    wait_scatter((nchunks + 1) % 2)
    plsc.subcore_barrier()
    pltpu.sync_copy(aggr_sh.at[pl.ds(s * WR, WR)],
                    out_hbm.at[pl.ds(c * N + s * WR, WR)])

    @pl.when(s == 0)
    def _():
        pltpu.sync_copy(aggr_sh.at[pl.ds(NS * WR, N - NS * WR)],
                        out_hbm.at[pl.ds(c * N + NS * WR, N - NS * WR)])


@functools.cache
def _edge_stage(esplit):
    return functools.partial(
        pl.kernel,
        out_type=jax.ShapeDtypeStruct((2 * N, HC), jnp.float32),
        mesh=_MESH,
        scratch_types=(
            [pltpu.VMEM((K,), jnp.int32)] * 4
            + [pltpu.VMEM((K, HC), jnp.float32)] * 2
            + [pltpu.VMEM((K,), jnp.int32)] * 4
            + [pltpu.VMEM((K, HC), jnp.float32)] * 2
            + [pltpu.VMEM_SHARED((N, HC), jnp.float32)]
            + [pltpu.SemaphoreType.DMA] * 8
        ),
    )(functools.partial(_edge_body, esplit))


EB = 2000


def _eaw_body(ea_ref, dv_ref, w1_ref, l1_ref, b1_ref, w2_ref, l2_ref, b2_ref,
              w3_ref, l3_ref, b3_ref, o1_ref, o2_ref, o3_ref):
    ea = ea_ref[...]
    dv = dv_ref[...]
    for w_ref, lw_ref, b_ref, o_ref, split in (
            (w1_ref, l1_ref, b1_ref, o1_ref, False),
            (w2_ref, l2_ref, b2_ref, o2_ref, True),
            (w3_ref, l3_ref, b3_ref, o3_ref, True)):
        res = (jnp.dot(ea, w_ref[...], preferred_element_type=jnp.float32)
               + dv * lw_ref[...] + b_ref[...])
        if split:
            o_ref[0] = res[:, :HC]
            o_ref[1] = res[:, HC:]
        else:
            o_ref[...] = res


def _eaw_all(edge_attr, dosd_vals, params):
    p = params
    wspecs = []
    wargs = []
    for l in (1, 2, 3):
        we = p[f"We{l}"]
        cin = we.shape[1]
        wargs += [we[:17], we[17].reshape(1, cin), p[f"be{l}"].reshape(1, cin)]
        wspecs += [pl.BlockSpec((17, cin), lambda i: (0, 0)),
                   pl.BlockSpec((1, cin), lambda i: (0, 0)),
                   pl.BlockSpec((1, cin), lambda i: (0, 0))]
    return pl.pallas_call(
        _eaw_body,
        grid=(E // EB,),
        in_specs=[pl.BlockSpec((EB, 17), lambda i: (i, 0)),
                  pl.BlockSpec((EB, 1), lambda i: (i, 0))] + wspecs,
        out_specs=[pl.BlockSpec((EB, 128), lambda i: (i, 0)),
                   pl.BlockSpec((2, EB, 128), lambda i: (0, i, 0)),
                   pl.BlockSpec((2, EB, 128), lambda i: (0, i, 0))],
        out_shape=[jax.ShapeDtypeStruct((E, 128), jnp.float32),
                   jax.ShapeDtypeStruct((2, E, 128), jnp.float32),
                   jax.ShapeDtypeStruct((2, E, 128), jnp.float32)],
    )(edge_attr, dosd_vals, *wargs)


NB = 1000


def _node_body(first, last, alo_ref, ahi_ref, hlo_ref, hhi_ref, xa_ref,
               wg_ref, bg_ref, wa_ref, ba_ref, wb_ref, bb_ref, wf_ref,
               bf_ref, o_ref):
    gt = jnp.dot(xa_ref[...], wg_ref[...],
                 preferred_element_type=jnp.float32) + bg_ref[...]
    if first:
        inp = alo_ref[...] + ahi_ref[...] + hlo_ref[...] + gt
    else:
        inp = jnp.concatenate(
            [alo_ref[...] + hlo_ref[...], ahi_ref[...] + hhi_ref[...]],
            axis=1) + gt
    bf = jnp.bfloat16
    t1 = jax.nn.relu(jnp.dot(inp.astype(bf), wa_ref[...].astype(bf),
                             preferred_element_type=jnp.float32) + ba_ref[...])
    t2 = jax.nn.relu(jnp.dot(t1.astype(bf), wb_ref[...].astype(bf),
                             preferred_element_type=jnp.float32) + bb_ref[...])
    h3 = jnp.dot(t2.astype(bf), wf_ref[...].astype(bf),
                 preferred_element_type=jnp.float32) + bf_ref[...]
    if not last:
        h3 = jax.nn.relu(h3)
        o_ref[0] = h3[:, :128]
        o_ref[1] = h3[:, 128:]
    else:
        o_ref[...] = h3


def _node_mlp(l, aggr_flat, h_flat, xA, params):
    p = params
    cin = NNFEAT if l == 1 else H
    first = l == 1
    hcin = cin if first else cin // 2
    last = l == 3
    nblk = N // NB
    args = [aggr_flat, aggr_flat, h_flat, h_flat,
            xA.reshape(1, 21), p[f"Wg{l}"], p[f"bg{l}"].reshape(1, cin),
            p[f"W{l}a"], p[f"b{l}a"].reshape(1, H),
            p[f"W{l}b"], p[f"b{l}b"].reshape(1, OUT),
            p[f"Wf{l}"], p[f"bf{l}"].reshape(1, H)]
    in_specs = [
        pl.BlockSpec((NB, HC), lambda i: (i, 0)),
        pl.BlockSpec((NB, HC), lambda i: (i + nblk, 0)),
        pl.BlockSpec((NB, hcin), lambda i: (i, 0)),
        pl.BlockSpec((NB, hcin), lambda i: (i, 0) if first
                     else (i + nblk, 0)),
        pl.BlockSpec((1, 21), lambda i: (0, 0)),
        pl.BlockSpec((21, cin), lambda i: (0, 0)),
        pl.BlockSpec((1, cin), lambda i: (0, 0)),
        pl.BlockSpec((cin, H), lambda i: (0, 0)),
        pl.BlockSpec((1, H), lambda i: (0, 0)),
        pl.BlockSpec((H, OUT), lambda i: (0, 0)),
        pl.BlockSpec((1, OUT), lambda i: (0, 0)),
        pl.BlockSpec((OUT, H), lambda i: (0, 0)),
        pl.BlockSpec((1, H), lambda i: (0, 0)),
    ]
    if last:
        out_spec = pl.BlockSpec((NB, H), lambda i: (i, 0))
        out_shape = jax.ShapeDtypeStruct((N, H), jnp.float32)
    else:
        out_spec = pl.BlockSpec((2, NB, 128), lambda i: (0, i, 0))
        out_shape = jax.ShapeDtypeStruct((2, N, 128), jnp.float32)
    return pl.pallas_call(
        functools.partial(_node_body, first, last),
        grid=(nblk,),
        in_specs=in_specs,
        out_specs=out_spec,
        out_shape=out_shape,
    )(*args)


def _pool_body(h_ref, b_ref, ps_ref, cnt_ref):
    i = pl.program_id(0)
    mask = (lax.broadcasted_iota(jnp.int32, (NUM_GRAPHS, NB), 0)
            == b_ref[0]).astype(jnp.float32)
    ps = jnp.dot(mask, h_ref[...], preferred_element_type=jnp.float32)
    cnt = jnp.sum(mask, axis=1, keepdims=True)

    @pl.when(i == 0)
    def _():
        ps_ref[...] = jnp.zeros_like(ps_ref)
        cnt_ref[...] = jnp.zeros_like(cnt_ref)

    ps_ref[...] += ps
    cnt_ref[...] += cnt


def _pool(h, batch_row):
    return pl.pallas_call(
        _pool_body,
        grid=(N // NB,),
        in_specs=[pl.BlockSpec((NB, H), lambda i: (i, 0)),
                  pl.BlockSpec((1, 1, NB), lambda i: (i, 0, 0))],
        out_specs=[pl.BlockSpec((NUM_GRAPHS, H), lambda i: (0, 0)),
                   pl.BlockSpec((NUM_GRAPHS, 1), lambda i: (0, 0))],
        out_shape=[jax.ShapeDtypeStruct((NUM_GRAPHS, H), jnp.float32),
                   jax.ShapeDtypeStruct((NUM_GRAPHS, 1), jnp.float32)],
    )(h, batch_row)


def _head_body(ps_ref, cnt_ref, w_ref, b_ref, o_ref):
    pooled = ps_ref[...] / jnp.maximum(cnt_ref[...], 1.0)
    o_ref[...] = jax.nn.sigmoid(
        jnp.dot(pooled, w_ref[...], preferred_element_type=jnp.float32)
        + b_ref[...]) * 0.5


def _head(ps, cnt, wfc, bfc):
    return pl.pallas_call(
        _head_body,
        out_shape=jax.ShapeDtypeStruct((NUM_GRAPHS, 1), jnp.float32),
    )(ps, cnt, wfc, bfc.reshape(1, 1))


def kernel(x, edge_index, edge_attr, xA, dosd_distances, batch, params):
    p = params
    src = edge_index[0]
    dst = edge_index[1]

    flat = src * N + dst
    dosd_vals = _dosd_gather(dosd_distances.reshape(N * N), flat)

    eaw1, eaw2, eaw3 = _eaw_all(edge_attr, dosd_vals.reshape(E, 1), p)

    batch_row = batch.reshape(N // NB, 1, NB)

    h_flat = x
    for l, eaw in ((1, eaw1), (2, eaw2), (3, eaw3)):
        esplit = l == 1
        aggr_flat = _edge_stage(esplit)(h_flat, eaw.reshape(-1, HC), src, dst)
        if l < 3:
            h_flat = _node_mlp(l, aggr_flat, h_flat, xA, p).reshape(2 * N, HC)
        else:
            h_final = _node_mlp(l, aggr_flat, h_flat, xA, p)
    ps, cnt = _pool(h_final, batch_row)
    return _head(ps, cnt, p["Wfc"], p["bfc"])
